# baseline (speedup 1.0000x reference)
import functools

import jax
import jax.numpy as jnp
from jax import lax
from jax.experimental import pallas as pl
from jax.experimental.pallas import tpu as pltpu
from jax.experimental.pallas import tpu_sc as plsc

_VOCAB = 1000
_MAX_LEN = 20
_HIDDEN = 32
_BATCH = 1024
_IN_F = _MAX_LEN * _HIDDEN
_OUT_F = _MAX_LEN * _VOCAB

_NC, _NS = 2, 16
_NW = _NC * _NS
_B_TOTAL = _BATCH * _MAX_LEN
_B_PER_W = _B_TOTAL // _NW


_PAD_W = 128


def _sc_gather(table_pad, idx_flat):
    mesh = plsc.VectorSubcoreMesh(core_axis_name="c", subcore_axis_name="s")

    @functools.partial(
        pl.kernel,
        mesh=mesh,
        out_type=jax.ShapeDtypeStruct((_B_TOTAL, _PAD_W), jnp.float32),
        scratch_types=[
            pltpu.VMEM((_B_PER_W,), jnp.int32),
            pltpu.VMEM((_B_PER_W, _PAD_W), jnp.float32),
            pltpu.SemaphoreType.DMA,
        ],
    )
    def k(table_hbm, idx_hbm, out_hbm, idx_v, rows_v, sem):
        wid = lax.axis_index("s") * _NC + lax.axis_index("c")
        base = wid * _B_PER_W
        pltpu.sync_copy(idx_hbm.at[pl.ds(base, _B_PER_W)], idx_v)
        pltpu.async_copy(table_hbm.at[idx_v], rows_v, sem).wait()
        pltpu.sync_copy(rows_v, out_hbm.at[pl.ds(base, _B_PER_W)])

    return k(table_pad, idx_flat)


_BN = 2048
_GRID_N = pl.cdiv(_OUT_F, _BN)


def _mm_body(flat_ref, w_ref, b_ref, out_ref):
    acc = lax.dot_general(
        flat_ref[...], w_ref[...],
        (((1,), (1,)), ((), ())),
        preferred_element_type=jnp.float32,
        precision=lax.Precision.HIGHEST,
    )
    out_ref[...] = acc + b_ref[...]


def _projection(flat, W, b_row):
    return pl.pallas_call(
        _mm_body,
        grid=(_GRID_N,),
        in_specs=[
            pl.BlockSpec((_BATCH, _IN_F), lambda j: (0, 0)),
            pl.BlockSpec((_BN, _IN_F), lambda j: (j, 0)),
            pl.BlockSpec((1, _BN), lambda j: (0, j)),
        ],
        out_specs=pl.BlockSpec((_BATCH, _BN), lambda j: (0, j)),
        out_shape=jax.ShapeDtypeStruct((_BATCH, _OUT_F), jnp.float32),
        compiler_params=pltpu.CompilerParams(
            dimension_semantics=("arbitrary",),
        ),
    )(flat, W, b_row)


def kernel(x, embed_table, W, b):
    idx = x.reshape(-1).astype(jnp.int32)
    table_pad = jnp.pad(embed_table, ((0, 0), (0, _PAD_W - _HIDDEN)))
    rows = _sc_gather(table_pad, idx)
    flat = rows[:, :_HIDDEN].reshape(_BATCH, _IN_F)
    out = _projection(flat, W, b.reshape(1, _OUT_F))
    return out.reshape(_BATCH, _MAX_LEN, _VOCAB)


# speedup vs baseline: 1.9581x; 1.9581x over previous
import functools

import jax
import jax.numpy as jnp
from jax import lax
from jax.experimental import pallas as pl
from jax.experimental.pallas import tpu as pltpu
from jax.experimental.pallas import tpu_sc as plsc

_VOCAB = 1000
_MAX_LEN = 20
_HIDDEN = 32
_BATCH = 1024
_IN_F = _MAX_LEN * _HIDDEN
_OUT_F = _MAX_LEN * _VOCAB

_NC, _NS = 2, 16
_NW = _NC * _NS
_B_TOTAL = _BATCH * _MAX_LEN
_B_PER_W = _B_TOTAL // _NW


_PAD_W = 128


def _sc_gather(table_pad, idx_flat):
    mesh = plsc.VectorSubcoreMesh(core_axis_name="c", subcore_axis_name="s")

    @functools.partial(
        pl.kernel,
        mesh=mesh,
        out_type=jax.ShapeDtypeStruct((_B_TOTAL, _PAD_W), jnp.float32),
        scratch_types=[
            pltpu.VMEM((_B_PER_W,), jnp.int32),
            pltpu.VMEM((_B_PER_W, _PAD_W), jnp.float32),
            pltpu.SemaphoreType.DMA,
        ],
    )
    def k(table_hbm, idx_hbm, out_hbm, idx_v, rows_v, sem):
        wid = lax.axis_index("s") * _NC + lax.axis_index("c")
        base = wid * _B_PER_W
        pltpu.sync_copy(idx_hbm.at[pl.ds(base, _B_PER_W)], idx_v)
        pltpu.async_copy(table_hbm.at[idx_v], rows_v, sem).wait()
        pltpu.sync_copy(rows_v, out_hbm.at[pl.ds(base, _B_PER_W)])

    return k(table_pad, idx_flat)


_BN = 2048
_GRID_N = pl.cdiv(_OUT_F, _BN)


def _mm_body(flat_ref, w_ref, b_ref, out_ref):
    acc = lax.dot_general(
        flat_ref[...], w_ref[...],
        (((1,), (1,)), ((), ())),
        preferred_element_type=jnp.float32,
        precision=lax.Precision.DEFAULT,
    )
    out_ref[...] = acc + b_ref[...]


def _projection(flat, W, b_row):
    return pl.pallas_call(
        _mm_body,
        grid=(_GRID_N,),
        in_specs=[
            pl.BlockSpec((_BATCH, _IN_F), lambda j: (0, 0)),
            pl.BlockSpec((_BN, _IN_F), lambda j: (j, 0)),
            pl.BlockSpec((1, _BN), lambda j: (0, j)),
        ],
        out_specs=pl.BlockSpec((_BATCH, _BN), lambda j: (0, j)),
        out_shape=jax.ShapeDtypeStruct((_BATCH, _OUT_F), jnp.float32),
        compiler_params=pltpu.CompilerParams(
            dimension_semantics=("arbitrary",),
        ),
    )(flat, W, b_row)


def kernel(x, embed_table, W, b):
    idx = x.reshape(-1).astype(jnp.int32)
    table_pad = jnp.pad(embed_table, ((0, 0), (0, _PAD_W - _HIDDEN)))
    rows = _sc_gather(table_pad, idx)
    flat = rows[:, :_HIDDEN].reshape(_BATCH, _IN_F)
    out = _projection(flat, W, b.reshape(1, _OUT_F))
    return out.reshape(_BATCH, _MAX_LEN, _VOCAB)
